# Initial kernel scaffold; baseline (speedup 1.0000x reference)
import jax
import jax.numpy as jnp
from jax.experimental import pallas as pl


def kernel(inputs, vocabulary):
    raise NotImplementedError("write your pallas kernel here")


# speedup vs baseline: 206.0661x; 206.0661x over previous
import functools

import jax
import jax.numpy as jnp
from jax import lax
from jax.experimental import pallas as pl
from jax.experimental.pallas import tpu as pltpu
from jax.experimental.pallas import tpu_sc as plsc

_NC = 2
_NS = 16
_L = 16
_NW = _NC * _NS

_KEY_RANGE = 1100
_INV_PAD = 1104


@functools.partial(jax.jit, static_argnames=("n", "vocab_size", "chunk"))
def _lookup(flat_inputs, vocabulary, *, n, vocab_size, chunk):
    per_w = n // _NW
    n_chunks = per_w // chunk

    mesh = plsc.VectorSubcoreMesh(
        core_axis_name="c", subcore_axis_name="s",
        num_cores=_NC, num_subcores=_NS)

    @functools.partial(
        pl.kernel,
        out_type=jax.ShapeDtypeStruct((n,), jnp.int32),
        mesh=mesh,
        compiler_params=pltpu.CompilerParams(needs_layout_passes=False),
        scratch_types=[
            pltpu.VMEM((vocab_size,), jnp.int32),
            pltpu.VMEM((_INV_PAD,), jnp.int32),
            pltpu.VMEM((chunk,), jnp.int32),
            pltpu.VMEM((chunk,), jnp.int32),
        ],
    )
    def k(in_hbm, vocab_hbm, out_hbm, vocab_v, inv_v, in_v, out_v):
        wid = lax.axis_index("s") * _NC + lax.axis_index("c")
        base = wid * per_w

        pltpu.sync_copy(vocab_hbm, vocab_v)

        def init_body(i, _):
            inv_v[pl.ds(i * _L, _L)] = jnp.full((_L,), vocab_size, jnp.int32)
            return 0
        lax.fori_loop(0, _INV_PAD // _L, init_body, 0)

        def scat_body(i, _):
            b = i * _L
            keys = vocab_v[pl.ds(b, _L)]
            plsc.store_scatter(inv_v, (keys,), b + lax.iota(jnp.int32, _L))
            return 0
        lax.fori_loop(0, vocab_size // _L, scat_body, 0)
        rem = vocab_size % _L
        if rem:
            b = vocab_size - _L
            keys = vocab_v[pl.ds(b, _L)]
            plsc.store_scatter(inv_v, (keys,), b + lax.iota(jnp.int32, _L))

        def chunk_body(ci, _):
            off = base + ci * chunk
            pltpu.sync_copy(in_hbm.at[pl.ds(off, chunk)], in_v)

            def gather_body(i, _):
                idx = in_v[pl.ds(i * _L, _L)]
                out_v[pl.ds(i * _L, _L)] = plsc.load_gather(inv_v, (idx,))
                return 0
            lax.fori_loop(0, chunk // _L, gather_body, 0)

            pltpu.sync_copy(out_v, out_hbm.at[pl.ds(off, chunk)])
            return 0
        lax.fori_loop(0, n_chunks, chunk_body, 0)

    return k(flat_inputs, vocabulary)


def kernel(inputs, vocabulary):
    n = inputs.size
    flat = inputs.reshape((n,))
    out = _lookup(flat, vocabulary,
                  n=n, vocab_size=vocabulary.shape[0], chunk=12800)
    return out.reshape(inputs.shape)


# speedup vs baseline: 339.7550x; 1.6488x over previous
import functools

import jax
import jax.numpy as jnp
from jax import lax
from jax.experimental import pallas as pl
from jax.experimental.pallas import tpu as pltpu
from jax.experimental.pallas import tpu_sc as plsc

_NC = 2
_NS = 16
_L = 16
_NW = _NC * _NS

_INV_PAD = 1104


@functools.partial(jax.jit, static_argnames=("vocab_size", "row_chunk"))
def _lookup(inputs, vocabulary, *, vocab_size, row_chunk):
    nrows, ncols = inputs.shape
    rows_per_w = nrows // _NW
    n_chunks = rows_per_w // row_chunk
    col_starts = list(range(0, ncols - _L + 1, _L))
    if col_starts[-1] + _L < ncols:
        col_starts.append(ncols - _L)

    mesh = plsc.VectorSubcoreMesh(
        core_axis_name="c", subcore_axis_name="s",
        num_cores=_NC, num_subcores=_NS)

    @functools.partial(
        pl.kernel,
        out_type=jax.ShapeDtypeStruct((nrows, ncols), jnp.int32),
        mesh=mesh,
        compiler_params=pltpu.CompilerParams(
            needs_layout_passes=False, use_tc_tiling_on_sc=True),
        scratch_types=[
            pltpu.VMEM((vocab_size,), jnp.int32),
            pltpu.VMEM((_INV_PAD,), jnp.int32),
            pltpu.VMEM((row_chunk, ncols), jnp.int32),
            pltpu.VMEM((row_chunk, ncols), jnp.int32),
        ],
    )
    def k(in_hbm, vocab_hbm, out_hbm, vocab_v, inv_v, in_v, out_v):
        wid = lax.axis_index("s") * _NC + lax.axis_index("c")
        base = wid * rows_per_w

        pltpu.sync_copy(vocab_hbm, vocab_v)

        def init_body(i, _):
            inv_v[pl.ds(i * _L, _L)] = jnp.full((_L,), vocab_size, jnp.int32)
            return 0
        lax.fori_loop(0, _INV_PAD // _L, init_body, 0)

        def scat_body(i, _):
            b = i * _L
            keys = vocab_v[pl.ds(b, _L)]
            plsc.store_scatter(inv_v, (keys,), b + lax.iota(jnp.int32, _L))
            return 0
        lax.fori_loop(0, vocab_size // _L, scat_body, 0)
        if vocab_size % _L:
            b = vocab_size - _L
            keys = vocab_v[pl.ds(b, _L)]
            plsc.store_scatter(inv_v, (keys,), b + lax.iota(jnp.int32, _L))

        def chunk_body(ci, _):
            r0 = base + ci * row_chunk
            pltpu.sync_copy(in_hbm.at[pl.ds(r0, row_chunk), :], in_v)

            def row_body(r, _):
                for c in col_starts:
                    idx = in_v[r, pl.ds(c, _L)]
                    out_v[r, pl.ds(c, _L)] = plsc.load_gather(inv_v, (idx,))
                return 0
            lax.fori_loop(0, row_chunk, row_body, 0)

            pltpu.sync_copy(out_v, out_hbm.at[pl.ds(r0, row_chunk), :])
            return 0
        lax.fori_loop(0, n_chunks, chunk_body, 0)

    return k(inputs, vocabulary)


def kernel(inputs, vocabulary):
    return _lookup(inputs, vocabulary,
                   vocab_size=vocabulary.shape[0], row_chunk=64)


# speedup vs baseline: 478.1535x; 1.4073x over previous
import functools

import jax
import jax.numpy as jnp
from jax import lax
from jax.experimental import pallas as pl
from jax.experimental.pallas import tpu as pltpu
from jax.experimental.pallas import tpu_sc as plsc

_NC = 2
_NS = 16
_L = 16
_NW = _NC * _NS

_INV_PAD = 1104


@functools.partial(jax.jit, static_argnames=("vocab_size", "row_chunk"))
def _lookup(inputs, vocabulary, *, vocab_size, row_chunk):
    nrows, ncols = inputs.shape
    rows_per_w = nrows // _NW
    n_chunks = rows_per_w // row_chunk
    col_starts = list(range(0, ncols - _L + 1, _L))
    if col_starts[-1] + _L < ncols:
        col_starts.append(ncols - _L)

    mesh = plsc.VectorSubcoreMesh(
        core_axis_name="c", subcore_axis_name="s",
        num_cores=_NC, num_subcores=_NS)

    @functools.partial(
        pl.kernel,
        out_type=jax.ShapeDtypeStruct((nrows, ncols), jnp.int32),
        mesh=mesh,
        compiler_params=pltpu.CompilerParams(
            needs_layout_passes=False, use_tc_tiling_on_sc=True),
        scratch_types=[
            pltpu.VMEM((vocab_size,), jnp.int32),
            pltpu.VMEM((_INV_PAD,), jnp.int32),
            pltpu.VMEM((row_chunk, ncols), jnp.int32),
            pltpu.VMEM((row_chunk, ncols), jnp.int32),
            pltpu.VMEM((row_chunk, ncols), jnp.int32),
            pltpu.VMEM((row_chunk, ncols), jnp.int32),
            pltpu.SemaphoreType.DMA,
            pltpu.SemaphoreType.DMA,
            pltpu.SemaphoreType.DMA,
            pltpu.SemaphoreType.DMA,
        ],
    )
    def k(in_hbm, vocab_hbm, out_hbm, vocab_v, inv_v,
          in0, in1, out0, out1, isem0, isem1, osem0, osem1):
        wid = lax.axis_index("s") * _NC + lax.axis_index("c")
        base = wid * rows_per_w

        pltpu.sync_copy(vocab_hbm, vocab_v)

        def init_body(i, _):
            inv_v[pl.ds(i * _L, _L)] = jnp.full((_L,), vocab_size, jnp.int32)
            return 0
        lax.fori_loop(0, _INV_PAD // _L, init_body, 0)

        def scat_body(i, _):
            b = i * _L
            keys = vocab_v[pl.ds(b, _L)]
            plsc.store_scatter(inv_v, (keys,), b + lax.iota(jnp.int32, _L))
            return 0
        lax.fori_loop(0, vocab_size // _L, scat_body, 0)
        if vocab_size % _L:
            b = vocab_size - _L
            keys = vocab_v[pl.ds(b, _L)]
            plsc.store_scatter(inv_v, (keys,), b + lax.iota(jnp.int32, _L))

        inb, outb = [in0, in1], [out0, out1]
        isem, osem = [isem0, isem1], [osem0, osem1]

        def start_in(ci):
            r0 = base + ci * row_chunk
            return pltpu.async_copy(
                in_hbm.at[pl.ds(r0, row_chunk), :], inb[ci % 2], isem[ci % 2])

        def start_out(ci):
            r0 = base + ci * row_chunk
            return pltpu.async_copy(
                outb[ci % 2], out_hbm.at[pl.ds(r0, row_chunk), :], osem[ci % 2])

        pending_in = [start_in(0), None]
        pending_out = [None, None]
        for ci in range(n_chunks):
            b = ci % 2
            if ci + 1 < n_chunks:
                pending_in[1 - b] = start_in(ci + 1)
            pending_in[b].wait()
            if pending_out[b] is not None:
                pending_out[b].wait()
            src, dst = inb[b], outb[b]

            @plsc.parallel_loop(0, row_chunk, step=1, unroll=2)
            def row_body(r):
                for c in col_starts:
                    idx = src[r, pl.ds(c, _L)]
                    dst[r, pl.ds(c, _L)] = plsc.load_gather(inv_v, (idx,))

            pending_out[b] = start_out(ci)
        for p in pending_out:
            if p is not None:
                p.wait()

    return k(inputs, vocabulary)


def kernel(inputs, vocabulary):
    return _lookup(inputs, vocabulary,
                   vocab_size=vocabulary.shape[0], row_chunk=64)
